# Initial kernel scaffold; baseline (speedup 1.0000x reference)
import jax
import jax.numpy as jnp
from jax.experimental import pallas as pl


def kernel(x, edge_index, batch_index, W1, b1, W2, b2, W_out, b_out):
    raise NotImplementedError("write your pallas kernel here")


# speedup vs baseline: 8.9335x; 8.9335x over previous
import functools

import jax
import jax.numpy as jnp
from jax import lax
from jax.experimental import pallas as pl
from jax.experimental.pallas import tpu as pltpu
from jax.experimental.pallas import tpu_sc as plsc

N = 10000
D = 128
E = 320000
G = 64
NCLS = 1000

NC, NS = 2, 16
NW = NC * NS
CH = 128
NCHUNK = 80
E_PAD = NW * NCHUNK * CH
N_PAD = 10112
ROWS_PT = N_PAD // NS
DEGW = 16

_MESH = plsc.VectorSubcoreMesh(
    core_axis_name="c", subcore_axis_name="s", num_cores=NC, num_subcores=NS
)


def _deg_body(dst_hbm, zeros_hbm, ones_hbm, out_hbm, dst_v, ones_v, acc):
    c = lax.axis_index("c")
    s = lax.axis_index("s")
    wid = s * NC + c
    r0 = s * ROWS_PT
    pltpu.sync_copy(zeros_hbm.at[pl.ds(r0, ROWS_PT)], acc.at[pl.ds(r0, ROWS_PT)])
    pltpu.sync_copy(ones_hbm, ones_v)
    pltpu.sync_copy(dst_hbm.at[pl.ds(wid * NCHUNK, NCHUNK)], dst_v)
    plsc.subcore_barrier()

    @pl.loop(0, NCHUNK)
    def _(j):
        pltpu.sync_copy(ones_v, acc.at[dst_v.at[j]], add=True)

    plsc.subcore_barrier()
    pltpu.sync_copy(acc.at[pl.ds(r0, ROWS_PT)], out_hbm.at[c, pl.ds(r0, ROWS_PT)])


_deg_call = pl.kernel(
    _deg_body,
    out_type=jax.ShapeDtypeStruct((NC, N_PAD, DEGW), jnp.float32),
    mesh=_MESH,
    scratch_types=[
        pltpu.VMEM((NCHUNK, CH), jnp.int32),
        pltpu.VMEM((CH, DEGW), jnp.float32),
        pltpu.VMEM_SHARED((N_PAD, DEGW), jnp.float32),
    ],
)


NH = NCHUNK // 2


def _spmm_body(hs_hbm, src_hbm, dst_hbm, zeros_hbm, out_hbm,
               src_v, dst_v, bufa, bufb, acc, sema, semb):
    c = lax.axis_index("c")
    s = lax.axis_index("s")
    wid = s * NC + c
    r0 = s * ROWS_PT
    pltpu.sync_copy(zeros_hbm.at[pl.ds(r0, ROWS_PT)], acc.at[pl.ds(r0, ROWS_PT)])
    plsc.subcore_barrier()

    @pl.loop(0, 2)
    def _(h):
        base = wid * NCHUNK + h * NH
        pltpu.sync_copy(src_hbm.at[pl.ds(base, NH)], src_v)
        pltpu.sync_copy(dst_hbm.at[pl.ds(base, NH)], dst_v)

        pltpu.make_async_copy(hs_hbm.at[src_v.at[0]], bufa, sema).start()

        @pl.loop(0, NH // 2)
        def _(i):
            j0 = 2 * i
            pltpu.make_async_copy(hs_hbm.at[src_v.at[j0]], bufa, sema).wait()
            pltpu.make_async_copy(hs_hbm.at[src_v.at[j0 + 1]], bufb, semb).start()
            pltpu.sync_copy(bufa, acc.at[dst_v.at[j0]], add=True)
            pltpu.make_async_copy(hs_hbm.at[src_v.at[j0 + 1]], bufb, semb).wait()
            jn = lax.rem(j0 + 2, NH)
            pltpu.make_async_copy(hs_hbm.at[src_v.at[jn]], bufa, sema).start()
            pltpu.sync_copy(bufb, acc.at[dst_v.at[j0 + 1]], add=True)

        pltpu.make_async_copy(hs_hbm.at[src_v.at[0]], bufa, sema).wait()

    plsc.subcore_barrier()
    pltpu.sync_copy(acc.at[pl.ds(r0, ROWS_PT)], out_hbm.at[c, pl.ds(r0, ROWS_PT)])


_spmm_call = pl.kernel(
    _spmm_body,
    out_type=jax.ShapeDtypeStruct((NC, N_PAD, D), jnp.float32),
    mesh=_MESH,
    scratch_types=[
        pltpu.VMEM((NH, CH), jnp.int32),
        pltpu.VMEM((NH, CH), jnp.int32),
        pltpu.VMEM((CH, D), jnp.float32),
        pltpu.VMEM((CH, D), jnp.float32),
        pltpu.VMEM_SHARED((N_PAD, D), jnp.float32),
        pltpu.SemaphoreType.DMA,
        pltpu.SemaphoreType.DMA,
    ],
)


_RB = 1000


def _dinv(dega_ref, degb_ref):
    deg = dega_ref[:, :1] + degb_ref[:, :1] + 1.0
    return lax.rsqrt(deg)


def _prep_body(x_ref, w1_ref, dega_ref, degb_ref, h1s_ref):
    h1 = jnp.dot(x_ref[...], w1_ref[...], preferred_element_type=jnp.float32)
    h1s_ref[...] = h1 * _dinv(dega_ref, degb_ref)


def _mid_body(sa_ref, sb_ref, h1s_ref, dega_ref, degb_ref, b1_ref, w2_ref,
              h2s_ref):
    dinv = _dinv(dega_ref, degb_ref)
    z1 = jnp.maximum(
        dinv * (sa_ref[...] + sb_ref[...] + h1s_ref[...]) + b1_ref[...], 0.0)
    h2 = jnp.dot(z1, w2_ref[...], preferred_element_type=jnp.float32)
    h2s_ref[...] = h2 * dinv


def _final_body(sa_ref, sb_ref, h2s_ref, dega_ref, degb_ref, b2_ref,
                bidx_ref, wout_ref, bout_ref, out_ref, pool_acc, cnt_acc):
    i = pl.program_id(0)
    dinv = _dinv(dega_ref, degb_ref)
    z2 = jnp.maximum(
        dinv * (sa_ref[...] + sb_ref[...] + h2s_ref[...]) + b2_ref[...], 0.0)
    gids = lax.broadcasted_iota(jnp.int32, (_RB, G), 1).astype(jnp.float32)
    oh = (bidx_ref[...] == gids).astype(jnp.float32)

    @pl.when(i == 0)
    def _():
        pool_acc[...] = jnp.zeros((G, D), jnp.float32)
        cnt_acc[...] = jnp.zeros((G, D), jnp.float32)

    dn = (((0,), (0,)), ((), ()))
    pool_acc[...] += lax.dot_general(oh, z2, dn,
                                     preferred_element_type=jnp.float32)
    cnt_acc[...] += lax.dot_general(oh, jnp.ones((_RB, D), jnp.float32), dn,
                                    preferred_element_type=jnp.float32)

    @pl.when(i == pl.num_programs(0) - 1)
    def _():
        pooled = pool_acc[...] / jnp.maximum(cnt_acc[...], 1.0)
        out_ref[...] = jnp.dot(pooled, wout_ref[...],
                               preferred_element_type=jnp.float32) + bout_ref[...]


_prep_call = pl.pallas_call(
    _prep_body,
    grid=(N // _RB,),
    in_specs=[
        pl.BlockSpec((_RB, D), lambda i: (i, 0)),
        pl.BlockSpec((D, D), lambda i: (0, 0)),
        pl.BlockSpec((_RB, DEGW), lambda i: (i, 0)),
        pl.BlockSpec((_RB, DEGW), lambda i: (i, 0)),
    ],
    out_specs=pl.BlockSpec((_RB, D), lambda i: (i, 0)),
    out_shape=jax.ShapeDtypeStruct((N, D), jnp.float32),
)

_mid_call = pl.pallas_call(
    _mid_body,
    grid=(N // _RB,),
    in_specs=[
        pl.BlockSpec((_RB, D), lambda i: (i, 0)),
        pl.BlockSpec((_RB, D), lambda i: (i, 0)),
        pl.BlockSpec((_RB, D), lambda i: (i, 0)),
        pl.BlockSpec((_RB, DEGW), lambda i: (i, 0)),
        pl.BlockSpec((_RB, DEGW), lambda i: (i, 0)),
        pl.BlockSpec((1, D), lambda i: (0, 0)),
        pl.BlockSpec((D, D), lambda i: (0, 0)),
    ],
    out_specs=pl.BlockSpec((_RB, D), lambda i: (i, 0)),
    out_shape=jax.ShapeDtypeStruct((N, D), jnp.float32),
)

_final_call = pl.pallas_call(
    _final_body,
    grid=(N // _RB,),
    in_specs=[
        pl.BlockSpec((_RB, D), lambda i: (i, 0)),
        pl.BlockSpec((_RB, D), lambda i: (i, 0)),
        pl.BlockSpec((_RB, D), lambda i: (i, 0)),
        pl.BlockSpec((_RB, DEGW), lambda i: (i, 0)),
        pl.BlockSpec((_RB, DEGW), lambda i: (i, 0)),
        pl.BlockSpec((1, D), lambda i: (0, 0)),
        pl.BlockSpec((_RB, 1), lambda i: (i, 0)),
        pl.BlockSpec((D, NCLS), lambda i: (0, 0)),
        pl.BlockSpec((1, NCLS), lambda i: (0, 0)),
    ],
    out_specs=pl.BlockSpec((G, NCLS), lambda i: (0, 0)),
    out_shape=jax.ShapeDtypeStruct((G, NCLS), jnp.float32),
    scratch_shapes=[
        pltpu.VMEM((G, D), jnp.float32),
        pltpu.VMEM((G, D), jnp.float32),
    ],
)


@jax.jit
def kernel(x, edge_index, batch_index, W1, b1, W2, b2, W_out, b_out):
    src = edge_index[0].astype(jnp.int32)
    dst = edge_index[1].astype(jnp.int32)
    pad_n = E_PAD - E
    src2d = jnp.concatenate(
        [src, jnp.zeros((pad_n,), jnp.int32)]).reshape(E_PAD // CH, CH)
    dst2d = jnp.concatenate(
        [dst, jnp.full((pad_n,), N, jnp.int32)]).reshape(E_PAD // CH, CH)

    zeros_deg = jnp.zeros((N_PAD, DEGW), jnp.float32)
    ones_deg = jnp.ones((CH, DEGW), jnp.float32)
    zeros_acc = jnp.zeros((N_PAD, D), jnp.float32)
    bidx = batch_index.astype(jnp.float32).reshape(N, 1)

    deg = _deg_call(dst2d, zeros_deg, ones_deg)
    dega, degb = deg[0], deg[1]

    h1s = _prep_call(x, W1, dega, degb)
    s1 = _spmm_call(h1s, src2d, dst2d, zeros_acc)
    h2s = _mid_call(s1[0], s1[1], h1s, dega, degb, b1.reshape(1, D), W2)
    s2 = _spmm_call(h2s, src2d, dst2d, zeros_acc)
    out = _final_call(s2[0], s2[1], h2s, dega, degb, b2.reshape(1, D),
                      bidx, W_out, b_out.reshape(1, NCLS))
    return out


# speedup vs baseline: 9.0913x; 1.0177x over previous
import functools

import jax
import jax.numpy as jnp
from jax import lax
from jax.experimental import pallas as pl
from jax.experimental.pallas import tpu as pltpu
from jax.experimental.pallas import tpu_sc as plsc

N = 10000
D = 128
E = 320000
G = 64
NCLS = 1000

NC, NS = 2, 16
NW = NC * NS
CH = 128
NCHUNK = 80
E_PAD = NW * NCHUNK * CH
N_PAD = 10112
ROWS_PT = N_PAD // NS
DEGW = 16

_MESH = plsc.VectorSubcoreMesh(
    core_axis_name="c", subcore_axis_name="s", num_cores=NC, num_subcores=NS
)


def _deg_body(dst_hbm, zeros_hbm, ones_hbm, out_hbm, dst_v, ones_v, acc):
    c = lax.axis_index("c")
    s = lax.axis_index("s")
    wid = s * NC + c
    r0 = s * ROWS_PT
    pltpu.sync_copy(zeros_hbm.at[pl.ds(r0, ROWS_PT)], acc.at[pl.ds(r0, ROWS_PT)])
    pltpu.sync_copy(ones_hbm, ones_v)
    pltpu.sync_copy(dst_hbm.at[pl.ds(wid * NCHUNK, NCHUNK)], dst_v)
    plsc.subcore_barrier()

    @pl.loop(0, NCHUNK)
    def _(j):
        pltpu.sync_copy(ones_v, acc.at[dst_v.at[j]], add=True)

    plsc.subcore_barrier()
    pltpu.sync_copy(acc.at[pl.ds(r0, ROWS_PT)], out_hbm.at[c, pl.ds(r0, ROWS_PT)])


_deg_call = pl.kernel(
    _deg_body,
    out_type=jax.ShapeDtypeStruct((NC, N_PAD, DEGW), jnp.float32),
    mesh=_MESH,
    scratch_types=[
        pltpu.VMEM((NCHUNK, CH), jnp.int32),
        pltpu.VMEM((CH, DEGW), jnp.float32),
        pltpu.VMEM_SHARED((N_PAD, DEGW), jnp.float32),
    ],
)


NH = NCHUNK // 2


def _spmm_body(hs_hbm, src_hbm, dst_hbm, zeros_hbm, out_hbm,
               src_v, dst_v, bufa, bufb, acc, sga, sgb):
    c = lax.axis_index("c")
    s = lax.axis_index("s")
    wid = s * NC + c
    r0 = s * ROWS_PT
    pltpu.sync_copy(zeros_hbm.at[pl.ds(r0, ROWS_PT)], acc.at[pl.ds(r0, ROWS_PT)])
    plsc.subcore_barrier()

    @pl.loop(0, 2)
    def _(h):
        base = wid * NCHUNK + h * NH
        pltpu.sync_copy(src_hbm.at[pl.ds(base, NH)], src_v)
        pltpu.sync_copy(dst_hbm.at[pl.ds(base, NH)], dst_v)

        def gat(j, buf, sem):
            return pltpu.make_async_copy(hs_hbm.at[src_v.at[j]], buf, sem)

        gat(0, bufa, sga).start()
        gat(1, bufb, sgb).start()

        @pl.loop(0, NH // 2)
        def _(i):
            j0 = 2 * i
            gat(j0, bufa, sga).wait()
            pltpu.sync_copy(bufa, acc.at[dst_v.at[j0]], add=True)
            gat(lax.rem(j0 + 2, NH), bufa, sga).start()
            gat(j0 + 1, bufb, sgb).wait()
            pltpu.sync_copy(bufb, acc.at[dst_v.at[j0 + 1]], add=True)
            gat(lax.rem(j0 + 3, NH), bufb, sgb).start()

        gat(0, bufa, sga).wait()
        gat(1, bufb, sgb).wait()

    plsc.subcore_barrier()
    pltpu.sync_copy(acc.at[pl.ds(r0, ROWS_PT)], out_hbm.at[c, pl.ds(r0, ROWS_PT)])


_spmm_call = pl.kernel(
    _spmm_body,
    out_type=jax.ShapeDtypeStruct((NC, N_PAD, D), jnp.float32),
    mesh=_MESH,
    scratch_types=[
        pltpu.VMEM((NH, CH), jnp.int32),
        pltpu.VMEM((NH, CH), jnp.int32),
        pltpu.VMEM((CH, D), jnp.float32),
        pltpu.VMEM((CH, D), jnp.float32),
        pltpu.VMEM_SHARED((N_PAD, D), jnp.float32),
        pltpu.SemaphoreType.DMA,
        pltpu.SemaphoreType.DMA,
    ],
)


_RB = 1000


def _dinv(dega_ref, degb_ref):
    deg = dega_ref[:, :1] + degb_ref[:, :1] + 1.0
    return lax.rsqrt(deg)


def _prep_body(x_ref, w1_ref, dega_ref, degb_ref, h1s_ref):
    h1 = jnp.dot(x_ref[...], w1_ref[...], preferred_element_type=jnp.float32)
    h1s_ref[...] = h1 * _dinv(dega_ref, degb_ref)


def _mid_body(sa_ref, sb_ref, h1s_ref, dega_ref, degb_ref, b1_ref, w2_ref,
              h2s_ref):
    dinv = _dinv(dega_ref, degb_ref)
    z1 = jnp.maximum(
        dinv * (sa_ref[...] + sb_ref[...] + h1s_ref[...]) + b1_ref[...], 0.0)
    h2 = jnp.dot(z1, w2_ref[...], preferred_element_type=jnp.float32)
    h2s_ref[...] = h2 * dinv


def _final_body(sa_ref, sb_ref, h2s_ref, dega_ref, degb_ref, b2_ref,
                bidx_ref, wout_ref, bout_ref, out_ref, pool_acc, cnt_acc):
    i = pl.program_id(0)
    dinv = _dinv(dega_ref, degb_ref)
    z2 = jnp.maximum(
        dinv * (sa_ref[...] + sb_ref[...] + h2s_ref[...]) + b2_ref[...], 0.0)
    gids = lax.broadcasted_iota(jnp.int32, (_RB, G), 1).astype(jnp.float32)
    oh = (bidx_ref[...] == gids).astype(jnp.float32)

    @pl.when(i == 0)
    def _():
        pool_acc[...] = jnp.zeros((G, D), jnp.float32)
        cnt_acc[...] = jnp.zeros((G, D), jnp.float32)

    dn = (((0,), (0,)), ((), ()))
    pool_acc[...] += lax.dot_general(oh, z2, dn,
                                     preferred_element_type=jnp.float32)
    cnt_acc[...] += lax.dot_general(oh, jnp.ones((_RB, D), jnp.float32), dn,
                                    preferred_element_type=jnp.float32)

    @pl.when(i == pl.num_programs(0) - 1)
    def _():
        pooled = pool_acc[...] / jnp.maximum(cnt_acc[...], 1.0)
        out_ref[...] = jnp.dot(pooled, wout_ref[...],
                               preferred_element_type=jnp.float32) + bout_ref[...]


_prep_call = pl.pallas_call(
    _prep_body,
    grid=(N // _RB,),
    in_specs=[
        pl.BlockSpec((_RB, D), lambda i: (i, 0)),
        pl.BlockSpec((D, D), lambda i: (0, 0)),
        pl.BlockSpec((_RB, DEGW), lambda i: (i, 0)),
        pl.BlockSpec((_RB, DEGW), lambda i: (i, 0)),
    ],
    out_specs=pl.BlockSpec((_RB, D), lambda i: (i, 0)),
    out_shape=jax.ShapeDtypeStruct((N, D), jnp.float32),
)

_mid_call = pl.pallas_call(
    _mid_body,
    grid=(N // _RB,),
    in_specs=[
        pl.BlockSpec((_RB, D), lambda i: (i, 0)),
        pl.BlockSpec((_RB, D), lambda i: (i, 0)),
        pl.BlockSpec((_RB, D), lambda i: (i, 0)),
        pl.BlockSpec((_RB, DEGW), lambda i: (i, 0)),
        pl.BlockSpec((_RB, DEGW), lambda i: (i, 0)),
        pl.BlockSpec((1, D), lambda i: (0, 0)),
        pl.BlockSpec((D, D), lambda i: (0, 0)),
    ],
    out_specs=pl.BlockSpec((_RB, D), lambda i: (i, 0)),
    out_shape=jax.ShapeDtypeStruct((N, D), jnp.float32),
)

_final_call = pl.pallas_call(
    _final_body,
    grid=(N // _RB,),
    in_specs=[
        pl.BlockSpec((_RB, D), lambda i: (i, 0)),
        pl.BlockSpec((_RB, D), lambda i: (i, 0)),
        pl.BlockSpec((_RB, D), lambda i: (i, 0)),
        pl.BlockSpec((_RB, DEGW), lambda i: (i, 0)),
        pl.BlockSpec((_RB, DEGW), lambda i: (i, 0)),
        pl.BlockSpec((1, D), lambda i: (0, 0)),
        pl.BlockSpec((_RB, 1), lambda i: (i, 0)),
        pl.BlockSpec((D, NCLS), lambda i: (0, 0)),
        pl.BlockSpec((1, NCLS), lambda i: (0, 0)),
    ],
    out_specs=pl.BlockSpec((G, NCLS), lambda i: (0, 0)),
    out_shape=jax.ShapeDtypeStruct((G, NCLS), jnp.float32),
    scratch_shapes=[
        pltpu.VMEM((G, D), jnp.float32),
        pltpu.VMEM((G, D), jnp.float32),
    ],
)


@jax.jit
def kernel(x, edge_index, batch_index, W1, b1, W2, b2, W_out, b_out):
    src = edge_index[0].astype(jnp.int32)
    dst = edge_index[1].astype(jnp.int32)
    pad_n = E_PAD - E
    src2d = jnp.concatenate(
        [src, jnp.zeros((pad_n,), jnp.int32)]).reshape(E_PAD // CH, CH)
    pad_dst = N + jnp.arange(pad_n, dtype=jnp.int32) % (N_PAD - N)
    dst2d = jnp.concatenate([dst, pad_dst]).reshape(E_PAD // CH, CH)

    zeros_deg = jnp.zeros((N_PAD, DEGW), jnp.float32)
    ones_deg = jnp.ones((CH, DEGW), jnp.float32)
    zeros_acc = jnp.zeros((N_PAD, D), jnp.float32)
    bidx = batch_index.astype(jnp.float32).reshape(N, 1)

    deg = _deg_call(dst2d, zeros_deg, ones_deg)
    dega, degb = deg[0], deg[1]

    h1s = _prep_call(x, W1, dega, degb)
    s1 = _spmm_call(h1s, src2d, dst2d, zeros_acc)
    h2s = _mid_call(s1[0], s1[1], h1s, dega, degb, b1.reshape(1, D), W2)
    s2 = _spmm_call(h2s, src2d, dst2d, zeros_acc)
    out = _final_call(s2[0], s2[1], h2s, dega, degb, b2.reshape(1, D),
                      bidx, W_out, b_out.reshape(1, NCLS))
    return out
